# baseline (speedup 1.0000x reference)
import jax
import jax.numpy as jnp
from jax.experimental import pallas as pl
from jax.experimental.pallas import tpu as pltpu

_TN = 131072
_TM_FALLBACK = 16384


def _lanedense_kernel(w_ref, xt_ref, ot_ref):
    ot_ref[...] = jax.lax.dot_general(
        w_ref[...],
        xt_ref[...],
        (((0,), (0,)), ((), ())),
        preferred_element_type=jnp.float32,
    )


def _rowtile_kernel(x_ref, w0_ref, w1_ref, o_ref):
    w = jnp.dot(w0_ref[...], w1_ref[...], preferred_element_type=jnp.float32)
    o_ref[...] = jnp.dot(x_ref[...], w, preferred_element_type=jnp.float32)


def _rowtile_path(x, pl0, weight1):
    n, k = x.shape
    n_out = weight1.shape[1]
    tm = min(_TM_FALLBACK, n)
    return pl.pallas_call(
        _rowtile_kernel,
        out_shape=jax.ShapeDtypeStruct((n, n_out), x.dtype),
        grid=(pl.cdiv(n, tm),),
        in_specs=[
            pl.BlockSpec((tm, k), lambda i: (i, 0)),
            pl.BlockSpec((k, pl0.shape[1]), lambda i: (0, 0)),
            pl.BlockSpec((weight1.shape[0], n_out), lambda i: (0, 0)),
        ],
        out_specs=pl.BlockSpec((tm, n_out), lambda i: (i, 0)),
        compiler_params=pltpu.CompilerParams(
            dimension_semantics=("parallel",),
            vmem_limit_bytes=100 << 20,
        ),
    )(x, pl0, weight1)


def kernel(x, pl0, pl1, weight1, weight2):
    n, k = x.shape
    n_out = weight1.shape[1]
    if n < 1024:
        return _rowtile_path(x, pl0, weight1)

    w = jnp.dot(pl0, weight1, preferred_element_type=jnp.float32)

    xt = x.T

    tn = min(_TN, n)
    cost = pl.CostEstimate(
        flops=2 * n * k * n_out,
        transcendentals=0,
        bytes_accessed=(n * k + n * n_out + k * n_out) * 4,
    )
    ot = pl.pallas_call(
        _lanedense_kernel,
        out_shape=jax.ShapeDtypeStruct((n_out, n), jnp.float32),
        grid=(pl.cdiv(n, tn),),
        in_specs=[
            pl.BlockSpec((n_out, k), lambda i: (0, 0)),
            pl.BlockSpec((k, tn), lambda i: (0, i)),
        ],
        out_specs=pl.BlockSpec((n_out, tn), lambda i: (0, i)),
        compiler_params=pltpu.CompilerParams(
            dimension_semantics=("parallel",),
            vmem_limit_bytes=100 << 20,
        ),
        cost_estimate=cost,
    )(w, xt)
    return ot.T


# speedup vs baseline: 1.0041x; 1.0041x over previous
import jax
import jax.numpy as jnp
from jax.experimental import pallas as pl
from jax.experimental.pallas import tpu as pltpu

_TN = 131072
_TM_FALLBACK = 16384


def _lanedense_kernel(wt_ref, xt_ref, ot_ref):
    ot_ref[...] = jnp.dot(
        wt_ref[...], xt_ref[...], preferred_element_type=jnp.float32
    )


def _rowtile_kernel(x_ref, w0_ref, w1_ref, o_ref):
    w = jnp.dot(w0_ref[...], w1_ref[...], preferred_element_type=jnp.float32)
    o_ref[...] = jnp.dot(x_ref[...], w, preferred_element_type=jnp.float32)


def _rowtile_path(x, pl0, weight1):
    n, k = x.shape
    n_out = weight1.shape[1]
    tm = min(_TM_FALLBACK, n)
    return pl.pallas_call(
        _rowtile_kernel,
        out_shape=jax.ShapeDtypeStruct((n, n_out), x.dtype),
        grid=(pl.cdiv(n, tm),),
        in_specs=[
            pl.BlockSpec((tm, k), lambda i: (i, 0)),
            pl.BlockSpec((k, pl0.shape[1]), lambda i: (0, 0)),
            pl.BlockSpec((weight1.shape[0], n_out), lambda i: (0, 0)),
        ],
        out_specs=pl.BlockSpec((tm, n_out), lambda i: (i, 0)),
        compiler_params=pltpu.CompilerParams(
            dimension_semantics=("parallel",),
            vmem_limit_bytes=100 << 20,
        ),
    )(x, pl0, weight1)


def kernel(x, pl0, pl1, weight1, weight2):
    n, k = x.shape
    n_out = weight1.shape[1]
    if n < 1024:
        return _rowtile_path(x, pl0, weight1)

    w = jnp.dot(pl0, weight1, preferred_element_type=jnp.float32)
    wt = w.T
    xt = x.T

    tn = min(_TN, n)
    cost = pl.CostEstimate(
        flops=2 * n * k * n_out,
        transcendentals=0,
        bytes_accessed=(n * k + n * n_out + k * n_out) * 4,
    )
    ot = pl.pallas_call(
        _lanedense_kernel,
        out_shape=jax.ShapeDtypeStruct((n_out, n), jnp.float32),
        grid=(pl.cdiv(n, tn),),
        in_specs=[
            pl.BlockSpec((n_out, k), lambda i: (0, 0)),
            pl.BlockSpec((k, tn), lambda i: (0, i)),
        ],
        out_specs=pl.BlockSpec((n_out, tn), lambda i: (0, i)),
        compiler_params=pltpu.CompilerParams(
            dimension_semantics=("parallel",),
            vmem_limit_bytes=100 << 20,
        ),
        cost_estimate=cost,
    )(wt, xt)
    return ot.T
